# Initial kernel scaffold; baseline (speedup 1.0000x reference)
import jax
import jax.numpy as jnp
from jax.experimental import pallas as pl


def kernel(edge_index, x, W, att_src, att_dst, bias):
    raise NotImplementedError("write your pallas kernel here")


# speedup vs baseline: 19.2610x; 19.2610x over previous
import functools

import jax
import jax.numpy as jnp
from jax import lax
from jax.experimental import pallas as pl
from jax.experimental.pallas import tpu as pltpu
import jax.experimental.pallas.tpu_sc as plsc

N_NODES = 10000
D = 128
E_EDGES = 320000
E_TOT = E_EDGES + N_NODES
NW = 32
B = 128
T_BATCH = 81
PER_W = B * T_BATCH
E_PAD = NW * PER_W
N_PAD = 10240
STRIPE = N_PAD // 16


def _tc_prologue(x_ref, w_ref, as_ref, ad_ref, h_ref, asum_ref, adsum_ref):
    h = jnp.dot(x_ref[...], w_ref[...], preferred_element_type=jnp.float32)
    h_ref[...] = h
    asum_ref[...] = jnp.sum(h * as_ref[...][None, :], axis=1, keepdims=True)
    adsum_ref[...] = jnp.sum(h * ad_ref[...][None, :], axis=1, keepdims=True)


def _sc_edges(src_hbm, dst_hbm, h_hbm, asrc_hbm, adst_hbm,
              u_out, s_out,
              src_v, dst_v, av_v, bv_v, p_v, rows_v, sem,
              u_sh, s_sh):
    c = lax.axis_index("c")
    s = lax.axis_index("s")
    wid = s * 2 + c

    def zero_rows(b, carry):
        for j in range(8):
            rows_v[b, pl.ds(j * 16, 16)] = jnp.zeros((16,), jnp.float32)
        return carry
    lax.fori_loop(0, B, zero_rows, 0)
    for k in range(STRIPE // B):
        pltpu.sync_copy(rows_v, u_sh.at[pl.ds(s * STRIPE + k * B, B)])
        pltpu.sync_copy(rows_v.at[0], s_sh.at[pl.ds(s * STRIPE + k * B, B)])
    plsc.subcore_barrier()

    def batch_body(t, carry):
        base = wid * PER_W + t * B
        pltpu.sync_copy(src_hbm.at[pl.ds(base, B)], src_v)
        pltpu.sync_copy(dst_hbm.at[pl.ds(base, B)], dst_v)
        pltpu.async_copy(asrc_hbm.at[src_v], av_v, sem).wait()
        pltpu.async_copy(adst_hbm.at[dst_v], bv_v, sem).wait()
        pltpu.async_copy(h_hbm.at[src_v], rows_v, sem).wait()
        for j in range(8):
            sl = pl.ds(j * 16, 16)
            a = av_v[sl] + bv_v[sl]
            e = jnp.where(a >= 0.0, a, 0.2 * a)
            p = jnp.exp(e)
            gid = base + j * 16 + lax.iota(jnp.int32, 16)
            p_v[sl] = jnp.where(gid < E_TOT, p, 0.0)

        def scale_group(g, carry):
            pg = p_v[pl.ds(g * 16, 16)]
            for l in range(16):
                pv = pg[l]
                row = g * 16 + l
                for j in range(8):
                    sl = pl.ds(j * 16, 16)
                    rows_v[row, sl] = rows_v[row, sl] * pv
            return carry
        lax.fori_loop(0, 8, scale_group, 0)

        pltpu.sync_copy(rows_v, u_sh.at[dst_v], add=True)
        pltpu.sync_copy(p_v, s_sh.at[dst_v], add=True)
        return carry
    lax.fori_loop(0, T_BATCH, batch_body, 0)

    plsc.subcore_barrier()
    pltpu.sync_copy(u_sh.at[pl.ds(s * STRIPE, STRIPE)],
                    u_out.at[c, pl.ds(s * STRIPE, STRIPE)])
    pltpu.sync_copy(s_sh.at[pl.ds(s * STRIPE, STRIPE)],
                    s_out.at[c, pl.ds(s * STRIPE, STRIPE)])


def _tc_epilogue(u_ref, s_ref, b_ref, o_ref):
    u = u_ref[0] + u_ref[1]
    den = s_ref[0] + s_ref[1] + 1e-16
    o = u / den + b_ref[...][None, :]
    o_ref[...] = jnp.where(o >= 0.0, o, 0.01 * o)


@jax.jit
def _run(src_p, dst_p, x, W, att_src, att_dst, bias):
    h, asum, adsum = pl.pallas_call(
        _tc_prologue,
        out_shape=[
            jax.ShapeDtypeStruct((N_NODES, D), jnp.float32),
            jax.ShapeDtypeStruct((N_NODES, 1), jnp.float32),
            jax.ShapeDtypeStruct((N_NODES, 1), jnp.float32),
        ],
    )(x, W, att_src, att_dst)

    mesh = plsc.VectorSubcoreMesh(core_axis_name="c", subcore_axis_name="s")
    sc = pl.kernel(
        _sc_edges,
        out_type=[
            jax.ShapeDtypeStruct((2, N_PAD, D), jnp.float32),
            jax.ShapeDtypeStruct((2, N_PAD), jnp.float32),
        ],
        mesh=mesh,
        scratch_types=[
            pltpu.VMEM((B,), jnp.int32),
            pltpu.VMEM((B,), jnp.int32),
            pltpu.VMEM((B,), jnp.float32),
            pltpu.VMEM((B,), jnp.float32),
            pltpu.VMEM((B,), jnp.float32),
            pltpu.VMEM((B, D), jnp.float32),
            pltpu.SemaphoreType.DMA,
            pltpu.VMEM_SHARED((N_PAD, D), jnp.float32),
            pltpu.VMEM_SHARED((N_PAD,), jnp.float32),
        ],
    )
    u2, s2 = sc(src_p, dst_p, h, asum.reshape(-1), adsum.reshape(-1))

    out_full = pl.pallas_call(
        _tc_epilogue,
        out_shape=jax.ShapeDtypeStruct((N_PAD, D), jnp.float32),
    )(u2, s2.reshape(2, N_PAD, 1), bias)
    return out_full[:N_NODES]


def kernel(edge_index, x, W, att_src, att_dst, bias):
    src = edge_index[0].astype(jnp.int32)
    dst = edge_index[1].astype(jnp.int32)
    loops = jnp.arange(N_NODES, dtype=jnp.int32)
    pad = jnp.zeros((E_PAD - E_TOT,), jnp.int32)
    src_p = jnp.concatenate([src, loops, pad])
    dst_p = jnp.concatenate([dst, loops, pad])
    return _run(src_p, dst_p, x, W, att_src, att_dst, bias)


# speedup vs baseline: 23.9310x; 1.2425x over previous
import functools

import jax
import jax.numpy as jnp
from jax import lax
from jax.experimental import pallas as pl
from jax.experimental.pallas import tpu as pltpu
import jax.experimental.pallas.tpu_sc as plsc

N_NODES = 10000
D = 128
E_EDGES = 320000
E_TOT = E_EDGES + N_NODES
NW = 32
B = 128
T_BATCH = 82
NI = T_BATCH // 2
PER_W = B * T_BATCH
E_PAD = NW * PER_W
T_ALL = E_PAD // B
N_PAD = 10240
STRIPE = N_PAD // 16


def _tc_prologue(x_ref, w_ref, as_ref, ad_ref, h_ref, asum_ref, adsum_ref):
    h = jnp.dot(x_ref[...], w_ref[...], preferred_element_type=jnp.float32)
    h_ref[...] = h
    asum_ref[...] = jnp.sum(h * as_ref[...][None, :], axis=1, keepdims=True)
    adsum_ref[...] = jnp.sum(h * ad_ref[...][None, :], axis=1, keepdims=True)


def _sc_edges(ipack_hbm, h_hbm, asrc_hbm, adst_hbm,
              u_out, s_out,
              idx_a, idx_b, srcv_a, srcv_b, dstv_a, dstv_b,
              av_a, av_b, bv_a, bv_b, p_a, p_b, rows_a, rows_b,
              sem_i_a, sem_i_b, sem_g_a, sem_g_b,
              u_sh, s_sh):
    c = lax.axis_index("c")
    s = lax.axis_index("s")
    wid = s * 2 + c
    row0 = wid * T_BATCH

    slot = {
        0: (idx_a, srcv_a, dstv_a, av_a, bv_a, p_a, rows_a, sem_i_a, sem_g_a),
        1: (idx_b, srcv_b, dstv_b, av_b, bv_b, p_b, rows_b, sem_i_b, sem_g_b),
    }

    def fire_idx(t, x):
        idx, _, _, _, _, _, _, sem_i, _ = slot[x]
        pltpu.async_copy(ipack_hbm.at[row0 + t], idx, sem_i)

    def fire_gathers(x):
        idx, srcv, dstv, av, bv, _, rows, sem_i, sem_g = slot[x]
        pltpu.make_async_copy(ipack_hbm.at[0], idx, sem_i).wait()
        for j in range(8):
            sl = pl.ds(j * 16, 16)
            srcv[sl] = idx[sl]
            dstv[sl] = idx[pl.ds(B + j * 16, 16)]
        pltpu.async_copy(asrc_hbm.at[srcv], av, sem_g)
        pltpu.async_copy(adst_hbm.at[dstv], bv, sem_g)
        pltpu.async_copy(h_hbm.at[srcv], rows, sem_g)

    def process(t, x):
        idx, srcv, dstv, av, bv, p_v, rows, sem_i, sem_g = slot[x]
        pltpu.make_async_copy(asrc_hbm.at[srcv], av, sem_g).wait()
        pltpu.make_async_copy(adst_hbm.at[dstv], bv, sem_g).wait()
        pltpu.make_async_copy(h_hbm.at[srcv], rows, sem_g).wait()
        base = row0 * B + t * B
        for j in range(8):
            sl = pl.ds(j * 16, 16)
            a = av[sl] + bv[sl]
            e = jnp.where(a >= 0.0, a, 0.2 * a)
            p = jnp.exp(e)
            gid = base + j * 16 + lax.iota(jnp.int32, 16)
            p_v[sl] = jnp.where(gid < E_TOT, p, 0.0)

        def scale_group(g, carry):
            pg = p_v[pl.ds(g * 16, 16)]
            for l in range(16):
                pv = pg[l]
                row = g * 16 + l
                for j in range(8):
                    sl = pl.ds(j * 16, 16)
                    rows[row, sl] = rows[row, sl] * pv
            return carry
        lax.fori_loop(0, 8, scale_group, 0)

        pltpu.sync_copy(rows, u_sh.at[dstv], add=True)
        pltpu.sync_copy(p_v, s_sh.at[dstv], add=True)

    def zero_rows(b, carry):
        for j in range(8):
            rows_a[b, pl.ds(j * 16, 16)] = jnp.zeros((16,), jnp.float32)
        return carry
    lax.fori_loop(0, B, zero_rows, 0)
    for k in range(STRIPE // B):
        pltpu.sync_copy(rows_a, u_sh.at[pl.ds(s * STRIPE + k * B, B)])
        pltpu.sync_copy(rows_a.at[0], s_sh.at[pl.ds(s * STRIPE + k * B, B)])
    plsc.subcore_barrier()

    fire_idx(0, 0)
    fire_idx(1, 1)
    fire_gathers(0)

    def body(i, carry):
        t = 2 * i
        fire_gathers(1)

        @pl.when(i < NI - 1)
        def _():
            fire_idx(t + 2, 0)
        process(t, 0)

        @pl.when(i < NI - 1)
        def _():
            fire_gathers(0)
            fire_idx(t + 3, 1)
        process(t + 1, 1)
        return carry

    lax.fori_loop(0, NI, body, 0)

    plsc.subcore_barrier()
    pltpu.sync_copy(u_sh.at[pl.ds(s * STRIPE, STRIPE)],
                    u_out.at[c, pl.ds(s * STRIPE, STRIPE)])
    pltpu.sync_copy(s_sh.at[pl.ds(s * STRIPE, STRIPE)],
                    s_out.at[c, pl.ds(s * STRIPE, STRIPE)])


def _tc_epilogue(u_ref, s_ref, b_ref, o_ref):
    u = u_ref[0] + u_ref[1]
    den = s_ref[0] + s_ref[1] + 1e-16
    o = u / den + b_ref[...][None, :]
    o_ref[...] = jnp.where(o >= 0.0, o, 0.01 * o)


@jax.jit
def _run(ipack, x, W, att_src, att_dst, bias):
    h, asum, adsum = pl.pallas_call(
        _tc_prologue,
        out_shape=[
            jax.ShapeDtypeStruct((N_NODES, D), jnp.float32),
            jax.ShapeDtypeStruct((N_NODES, 1), jnp.float32),
            jax.ShapeDtypeStruct((N_NODES, 1), jnp.float32),
        ],
    )(x, W, att_src, att_dst)

    mesh = plsc.VectorSubcoreMesh(core_axis_name="c", subcore_axis_name="s")
    sc = pl.kernel(
        _sc_edges,
        out_type=[
            jax.ShapeDtypeStruct((2, N_PAD, D), jnp.float32),
            jax.ShapeDtypeStruct((2, N_PAD), jnp.float32),
        ],
        mesh=mesh,
        scratch_types=[
            pltpu.VMEM((2 * B,), jnp.int32),
            pltpu.VMEM((2 * B,), jnp.int32),
            pltpu.VMEM((B,), jnp.int32),
            pltpu.VMEM((B,), jnp.int32),
            pltpu.VMEM((B,), jnp.int32),
            pltpu.VMEM((B,), jnp.int32),
            pltpu.VMEM((B,), jnp.float32),
            pltpu.VMEM((B,), jnp.float32),
            pltpu.VMEM((B,), jnp.float32),
            pltpu.VMEM((B,), jnp.float32),
            pltpu.VMEM((B,), jnp.float32),
            pltpu.VMEM((B,), jnp.float32),
            pltpu.VMEM((B, D), jnp.float32),
            pltpu.VMEM((B, D), jnp.float32),
            pltpu.SemaphoreType.DMA,
            pltpu.SemaphoreType.DMA,
            pltpu.SemaphoreType.DMA,
            pltpu.SemaphoreType.DMA,
            pltpu.VMEM_SHARED((N_PAD, D), jnp.float32),
            pltpu.VMEM_SHARED((N_PAD,), jnp.float32),
        ],
    )
    u2, s2 = sc(ipack, h, asum.reshape(-1), adsum.reshape(-1))

    out_full = pl.pallas_call(
        _tc_epilogue,
        out_shape=jax.ShapeDtypeStruct((N_PAD, D), jnp.float32),
    )(u2, s2.reshape(2, N_PAD, 1), bias)
    return out_full[:N_NODES]


def kernel(edge_index, x, W, att_src, att_dst, bias):
    src = edge_index[0].astype(jnp.int32)
    dst = edge_index[1].astype(jnp.int32)
    loops = jnp.arange(N_NODES, dtype=jnp.int32)
    pad = jnp.zeros((E_PAD - E_TOT,), jnp.int32)
    src_p = jnp.concatenate([src, loops, pad]).reshape(T_ALL, B)
    dst_p = jnp.concatenate([dst, loops, pad]).reshape(T_ALL, B)
    ipack = jnp.concatenate([src_p, dst_p], axis=1)
    return _run(ipack, x, W, att_src, att_dst, bias)


# speedup vs baseline: 50.9324x; 2.1283x over previous
import functools

import jax
import jax.numpy as jnp
from jax import lax
from jax.experimental import pallas as pl
from jax.experimental.pallas import tpu as pltpu
import jax.experimental.pallas.tpu_sc as plsc

N_NODES = 10000
D = 128
E_EDGES = 320000
E_TOT = E_EDGES + N_NODES
NW = 32
B = 128
T_BATCH = 82
NI = T_BATCH // 2
PER_W = B * T_BATCH
E_PAD = NW * PER_W
T_ALL = E_PAD // B
N_PAD = 10240
STRIPE = N_PAD // 16


def _tc_prologue(x_ref, w_ref, as_ref, ad_ref, h_ref, asum_ref, adsum_ref):
    h = jnp.dot(x_ref[...], w_ref[...], preferred_element_type=jnp.float32)
    h_ref[...] = h
    asum_ref[...] = jnp.sum(h * as_ref[...][None, :], axis=1, keepdims=True)
    adsum_ref[...] = jnp.sum(h * ad_ref[...][None, :], axis=1, keepdims=True)


def _sc_edges(ipack_hbm, h_hbm, asrc_hbm, adst_hbm,
              u_out, s_out,
              idx_a, idx_b, srcv_a, srcv_b, dstv_a, dstv_b,
              av_a, av_b, bv_a, bv_b, p_a, p_b, rows_a, rows_b,
              sem_i_a, sem_i_b, sem_g_a, sem_g_b,
              u_sh, s_sh):
    c = lax.axis_index("c")
    s = lax.axis_index("s")
    wid = s * 2 + c
    row0 = wid * T_BATCH

    slot = {
        0: (idx_a, srcv_a, dstv_a, av_a, bv_a, p_a, rows_a, sem_i_a, sem_g_a),
        1: (idx_b, srcv_b, dstv_b, av_b, bv_b, p_b, rows_b, sem_i_b, sem_g_b),
    }

    def fire_idx(t, x):
        idx, _, _, _, _, _, _, sem_i, _ = slot[x]
        pltpu.async_copy(ipack_hbm.at[row0 + t], idx, sem_i)

    def fire_gathers(x):
        idx, srcv, dstv, av, bv, _, rows, sem_i, sem_g = slot[x]
        pltpu.make_async_copy(ipack_hbm.at[0], idx, sem_i).wait()
        for j in range(8):
            sl = pl.ds(j * 16, 16)
            srcv[sl] = idx[sl]
            dstv[sl] = idx[pl.ds(B + j * 16, 16)]
        pltpu.async_copy(asrc_hbm.at[srcv], av, sem_g)
        pltpu.async_copy(adst_hbm.at[dstv], bv, sem_g)
        pltpu.async_copy(h_hbm.at[srcv], rows, sem_g)

    def process(t, x):
        idx, srcv, dstv, av, bv, p_v, rows, sem_i, sem_g = slot[x]
        pltpu.make_async_copy(asrc_hbm.at[srcv], av, sem_g).wait()
        pltpu.make_async_copy(adst_hbm.at[dstv], bv, sem_g).wait()
        pltpu.make_async_copy(h_hbm.at[srcv], rows, sem_g).wait()
        base = row0 * B + t * B
        for j in range(8):
            sl = pl.ds(j * 16, 16)
            a = av[sl] + bv[sl]
            e = jnp.where(a >= 0.0, a, 0.2 * a)
            p = jnp.exp(e)
            gid = base + j * 16 + lax.iota(jnp.int32, 16)
            p_v[sl] = jnp.where(gid < E_TOT, p, 0.0)

        def scale_group(g, carry):
            pg = p_v[pl.ds(g * 16, 16)]
            for l in range(16):
                pv = pg[l]
                row = g * 16 + l
                for j in range(8):
                    sl = pl.ds(j * 16, 16)
                    rows[row, sl] = rows[row, sl] * pv
            return carry
        lax.fori_loop(0, 8, scale_group, 0)

        pltpu.sync_copy(rows, u_sh.at[dstv], add=True)
        pltpu.sync_copy(p_v, s_sh.at[dstv], add=True)

    def zero_rows(b, carry):
        for j in range(8):
            rows_a[b, pl.ds(j * 16, 16)] = jnp.zeros((16,), jnp.float32)
        return carry
    lax.fori_loop(0, B, zero_rows, 0)
    for k in range(STRIPE // B):
        pltpu.sync_copy(rows_a, u_sh.at[pl.ds(s * STRIPE + k * B, B)])
        pltpu.sync_copy(rows_a.at[0], s_sh.at[pl.ds(s * STRIPE + k * B, B)])
    plsc.subcore_barrier()

    fire_idx(0, 0)
    fire_idx(1, 1)
    fire_gathers(0)

    def body(i, carry):
        t = 2 * i
        fire_gathers(1)

        @pl.when(i < NI - 1)
        def _():
            fire_idx(t + 2, 0)
        process(t, 0)

        @pl.when(i < NI - 1)
        def _():
            fire_gathers(0)
            fire_idx(t + 3, 1)
        process(t + 1, 1)
        return carry

    lax.fori_loop(0, NI, body, 0)

    plsc.subcore_barrier()
    pltpu.sync_copy(u_sh.at[pl.ds(s * STRIPE, STRIPE)],
                    u_out.at[c, pl.ds(s * STRIPE, STRIPE)])
    pltpu.sync_copy(s_sh.at[pl.ds(s * STRIPE, STRIPE)],
                    s_out.at[c, pl.ds(s * STRIPE, STRIPE)])


def _tc_epilogue(u_ref, s_ref, b_ref, o_ref):
    u = u_ref[0] + u_ref[1]
    den = s_ref[0] + s_ref[1] + 1e-16
    o = u / den + b_ref[...][None, :]
    o_ref[...] = jnp.where(o >= 0.0, o, 0.01 * o)


@jax.jit
def _run(ipack, x, W, att_src, att_dst, bias):
    h, asum, adsum = pl.pallas_call(
        _tc_prologue,
        out_shape=[
            jax.ShapeDtypeStruct((N_NODES, D), jnp.float32),
            jax.ShapeDtypeStruct((N_NODES, 1), jnp.float32),
            jax.ShapeDtypeStruct((N_NODES, 1), jnp.float32),
        ],
    )(x, W, att_src, att_dst)

    mesh = plsc.VectorSubcoreMesh(core_axis_name="c", subcore_axis_name="s")
    sc = pl.kernel(
        _sc_edges,
        out_type=[
            jax.ShapeDtypeStruct((2, N_PAD, D), jnp.float32),
            jax.ShapeDtypeStruct((2, N_PAD), jnp.float32),
        ],
        mesh=mesh,
        scratch_types=[
            pltpu.VMEM((2 * B,), jnp.int32),
            pltpu.VMEM((2 * B,), jnp.int32),
            pltpu.VMEM((B,), jnp.int32),
            pltpu.VMEM((B,), jnp.int32),
            pltpu.VMEM((B,), jnp.int32),
            pltpu.VMEM((B,), jnp.int32),
            pltpu.VMEM((B,), jnp.float32),
            pltpu.VMEM((B,), jnp.float32),
            pltpu.VMEM((B,), jnp.float32),
            pltpu.VMEM((B,), jnp.float32),
            pltpu.VMEM((B,), jnp.float32),
            pltpu.VMEM((B,), jnp.float32),
            pltpu.VMEM((B, D), jnp.float32),
            pltpu.VMEM((B, D), jnp.float32),
            pltpu.SemaphoreType.DMA,
            pltpu.SemaphoreType.DMA,
            pltpu.SemaphoreType.DMA,
            pltpu.SemaphoreType.DMA,
            pltpu.VMEM_SHARED((N_PAD, D), jnp.float32),
            pltpu.VMEM_SHARED((N_PAD,), jnp.float32),
        ],
    )
    u2, s2 = sc(ipack, h, asum.reshape(-1), adsum.reshape(-1))

    out_full = pl.pallas_call(
        _tc_epilogue,
        out_shape=jax.ShapeDtypeStruct((N_PAD, D), jnp.float32),
    )(u2, s2.reshape(2, N_PAD, 1), bias)
    return out_full[:N_NODES]


def kernel(edge_index, x, W, att_src, att_dst, bias):
    src = edge_index[0].astype(jnp.int32)
    dst = edge_index[1].astype(jnp.int32)
    loops = jnp.arange(N_NODES, dtype=jnp.int32)
    pad = jnp.arange(E_PAD - E_TOT, dtype=jnp.int32) % N_NODES
    src_p = jnp.concatenate([src, loops, pad]).reshape(T_ALL, B)
    dst_p = jnp.concatenate([dst, loops, pad]).reshape(T_ALL, B)
    ipack = jnp.concatenate([src_p, dst_p], axis=1)
    return _run(ipack, x, W, att_src, att_dst, bias)
